# baseline (speedup 1.0000x reference)
import jax
import jax.numpy as jnp
from jax.experimental import pallas as pl
from jax.experimental.pallas import tpu as pltpu

_N = 5000
_B = 128
_NB = 40
_NP = _NB * _B
_TH = 0.7


def _nms_body(x1c, y1c, x2c, y2c, x1r, y1r, x2r, y2r, out_ref, m_scr):
    out_ref[:, :] = jnp.zeros((_NB, _B), jnp.float32)
    lane = jax.lax.broadcasted_iota(jnp.int32, (1, _B), 1)
    jlt = (jax.lax.broadcasted_iota(jnp.int32, (_B, _B), 0)
           < jax.lax.broadcasted_iota(jnp.int32, (_B, _B), 1))

    def blk(b, carry):
        base = b * _B
        cx1 = x1c[pl.ds(base, _B), :]
        cy1 = y1c[pl.ds(base, _B), :]
        cx2 = x2c[pl.ds(base, _B), :]
        cy2 = y2c[pl.ds(base, _B), :]
        ac = (jnp.maximum(cx2 - cx1, 0.0) * jnp.maximum(cy2 - cy1, 0.0))

        rx1 = x1r[pl.ds(b, 1), :]
        ry1 = y1r[pl.ds(b, 1), :]
        rx2 = x2r[pl.ds(b, 1), :]
        ry2 = y2r[pl.ds(b, 1), :]
        ar = (jnp.maximum(rx2 - rx1, 0.0) * jnp.maximum(ry2 - ry1, 0.0))

        w = jnp.maximum(jnp.minimum(cx2, rx2) - jnp.maximum(cx1, rx1), 0.0)
        h = jnp.maximum(jnp.minimum(cy2, ry2) - jnp.maximum(cy1, ry1), 0.0)
        inter = w * h
        union = ac + ar - inter
        iou = inter / jnp.maximum(union, 1e-8)
        m_scr[:, :] = jnp.where((iou > _TH) & jlt, 1.0, 0.0)

        rem0 = out_ref[pl.ds(b, 1), :]

        def scan_body(i, rem):
            rowm = m_scr[pl.ds(i, 1), :]
            remi = pltpu.roll(rem, -i, 1)[:, 0:1]
            return jnp.maximum(rem, rowm * (1.0 - remi))

        rem = jax.lax.fori_loop(0, _B, scan_body, rem0)
        out_ref[pl.ds(b, 1), :] = rem
        keep = 1.0 - rem

        def cross(c, carry2):
            vx1 = x1r[pl.ds(c, 1), :]
            vy1 = y1r[pl.ds(c, 1), :]
            vx2 = x2r[pl.ds(c, 1), :]
            vy2 = y2r[pl.ds(c, 1), :]
            av = (jnp.maximum(vx2 - vx1, 0.0) * jnp.maximum(vy2 - vy1, 0.0))
            wv = jnp.maximum(jnp.minimum(cx2, vx2) - jnp.maximum(cx1, vx1), 0.0)
            hv = jnp.maximum(jnp.minimum(cy2, vy2) - jnp.maximum(cy1, vy1), 0.0)
            iv = wv * hv
            uv = ac + av - iv
            iouv = iv / jnp.maximum(uv, 1e-8)
            maskv = jnp.where(iouv > _TH, 1.0, 0.0)
            counts = jax.lax.dot_general(
                keep, maskv, (((1,), (0,)), ((), ())),
                preferred_element_type=jnp.float32)
            cur = out_ref[pl.ds(c, 1), :]
            out_ref[pl.ds(c, 1), :] = jnp.maximum(
                cur, jnp.where(counts >= 0.5, 1.0, 0.0))
            return carry2

        jax.lax.fori_loop(b + 1, _NB, cross, 0)
        return carry

    jax.lax.fori_loop(0, _NB, blk, 0)


def kernel(boxes, scores):
    order = jnp.argsort(-scores)
    b = jnp.take(boxes, order, axis=0)
    s = jnp.take(scores, order)
    bp = jnp.pad(b, ((0, _NP - _N), (0, 0)))
    x1c = bp[:, 0:1]
    y1c = bp[:, 1:2]
    x2c = bp[:, 2:3]
    y2c = bp[:, 3:4]
    x1r = bp[:, 0].reshape(_NB, _B)
    y1r = bp[:, 1].reshape(_NB, _B)
    x2r = bp[:, 2].reshape(_NB, _B)
    y2r = bp[:, 3].reshape(_NB, _B)
    removed = pl.pallas_call(
        _nms_body,
        out_shape=jax.ShapeDtypeStruct((_NB, _B), jnp.float32),
        scratch_shapes=[pltpu.VMEM((_B, _B), jnp.float32)],
    )(x1c, y1c, x2c, y2c, x1r, y1r, x2r, y2r)
    keep = 1.0 - removed.reshape(_NP)[:_N]
    proposals = jnp.concatenate([b, s[:, None]], axis=1) * keep[:, None]
    return proposals


# speedup vs baseline: 1.4942x; 1.4942x over previous
import jax
import jax.numpy as jnp
from jax.experimental import pallas as pl
from jax.experimental.pallas import tpu as pltpu

_N = 5000
_B = 128
_NB = 40
_NP = _NB * _B
_TH = 0.7


def _nms_body(x1c, y1c, x2c, y2c, x1r, y1r, x2r, y2r, out_ref, m_scr):
    out_ref[:, :] = jnp.zeros((_NB, _B), jnp.float32)
    lane = jax.lax.broadcasted_iota(jnp.int32, (1, _B), 1)
    jlt = (jax.lax.broadcasted_iota(jnp.int32, (_B, _B), 0)
           < jax.lax.broadcasted_iota(jnp.int32, (_B, _B), 1))

    def blk(b, carry):
        base = b * _B
        cx1 = x1c[pl.ds(base, _B), :]
        cy1 = y1c[pl.ds(base, _B), :]
        cx2 = x2c[pl.ds(base, _B), :]
        cy2 = y2c[pl.ds(base, _B), :]
        ac = (jnp.maximum(cx2 - cx1, 0.0) * jnp.maximum(cy2 - cy1, 0.0))

        rx1 = x1r[pl.ds(b, 1), :]
        ry1 = y1r[pl.ds(b, 1), :]
        rx2 = x2r[pl.ds(b, 1), :]
        ry2 = y2r[pl.ds(b, 1), :]
        ar = (jnp.maximum(rx2 - rx1, 0.0) * jnp.maximum(ry2 - ry1, 0.0))

        w = jnp.maximum(jnp.minimum(cx2, rx2) - jnp.maximum(cx1, rx1), 0.0)
        h = jnp.maximum(jnp.minimum(cy2, ry2) - jnp.maximum(cy1, ry1), 0.0)
        inter = w * h
        union = ac + ar - inter
        iou = inter / jnp.maximum(union, 1e-8)
        m_scr[:, :] = jnp.where((iou > _TH) & jlt, 1.0, 0.0)

        rem0 = out_ref[pl.ds(b, 1), :]

        def group(g, rem_rot):
            mg = pltpu.roll(m_scr[pl.ds(g * 8, 8), :], (_B - g * 8) % _B, 1)
            for k in range(8):
                row = mg[k:k + 1, :]
                flag = rem_rot[:, k:k + 1]
                rem_rot = jnp.maximum(rem_rot, row * (1.0 - flag))
            return pltpu.roll(rem_rot, _B - 8, 1)

        rem = jax.lax.fori_loop(0, _B // 8, group, rem0)
        out_ref[pl.ds(b, 1), :] = rem
        keep = 1.0 - rem

        def cross(c, carry2):
            vx1 = x1r[pl.ds(c, 1), :]
            vy1 = y1r[pl.ds(c, 1), :]
            vx2 = x2r[pl.ds(c, 1), :]
            vy2 = y2r[pl.ds(c, 1), :]
            av = (jnp.maximum(vx2 - vx1, 0.0) * jnp.maximum(vy2 - vy1, 0.0))
            wv = jnp.maximum(jnp.minimum(cx2, vx2) - jnp.maximum(cx1, vx1), 0.0)
            hv = jnp.maximum(jnp.minimum(cy2, vy2) - jnp.maximum(cy1, vy1), 0.0)
            iv = wv * hv
            uv = ac + av - iv
            iouv = iv / jnp.maximum(uv, 1e-8)
            maskv = jnp.where(iouv > _TH, 1.0, 0.0)
            counts = jax.lax.dot_general(
                keep, maskv, (((1,), (0,)), ((), ())),
                preferred_element_type=jnp.float32)
            cur = out_ref[pl.ds(c, 1), :]
            out_ref[pl.ds(c, 1), :] = jnp.maximum(
                cur, jnp.where(counts >= 0.5, 1.0, 0.0))
            return carry2

        jax.lax.fori_loop(b + 1, _NB, cross, 0)
        return carry

    jax.lax.fori_loop(0, _NB, blk, 0)


def kernel(boxes, scores):
    order = jnp.argsort(-scores)
    b = jnp.take(boxes, order, axis=0)
    s = jnp.take(scores, order)
    bp = jnp.pad(b, ((0, _NP - _N), (0, 0)))
    x1c = bp[:, 0:1]
    y1c = bp[:, 1:2]
    x2c = bp[:, 2:3]
    y2c = bp[:, 3:4]
    x1r = bp[:, 0].reshape(_NB, _B)
    y1r = bp[:, 1].reshape(_NB, _B)
    x2r = bp[:, 2].reshape(_NB, _B)
    y2r = bp[:, 3].reshape(_NB, _B)
    removed = pl.pallas_call(
        _nms_body,
        out_shape=jax.ShapeDtypeStruct((_NB, _B), jnp.float32),
        scratch_shapes=[pltpu.VMEM((_B, _B), jnp.float32)],
    )(x1c, y1c, x2c, y2c, x1r, y1r, x2r, y2r)
    keep = 1.0 - removed.reshape(_NP)[:_N]
    proposals = jnp.concatenate([b, s[:, None]], axis=1) * keep[:, None]
    return proposals


# speedup vs baseline: 1.6730x; 1.1197x over previous
import jax
import jax.numpy as jnp
from jax.experimental import pallas as pl
from jax.experimental.pallas import tpu as pltpu

_N = 5000
_B = 128
_NB = 40
_NP = _NB * _B
_TH = 0.7


def _nms_body(x1c, y1c, x2c, y2c, x1r, y1r, x2r, y2r,
              X1, Y1, X2, Y2, out_ref, m_scr, cross_ref):
    out_ref[:, :] = jnp.zeros((_NB, _B), jnp.float32)
    cross_ref[:, :] = jnp.zeros((1, _NP), jnp.float32)
    gl = jax.lax.broadcasted_iota(jnp.int32, (1, _NP), 1)
    jlt = (jax.lax.broadcasted_iota(jnp.int32, (_B, _B), 0)
           < jax.lax.broadcasted_iota(jnp.int32, (_B, _B), 1))
    VX1 = X1[:, :]
    VY1 = Y1[:, :]
    VX2 = X2[:, :]
    VY2 = Y2[:, :]
    AV = (jnp.maximum(VX2 - VX1, 0.0) * jnp.maximum(VY2 - VY1, 0.0))

    def blk(b, carry):
        base = b * _B
        cx1 = x1c[pl.ds(base, _B), :]
        cy1 = y1c[pl.ds(base, _B), :]
        cx2 = x2c[pl.ds(base, _B), :]
        cy2 = y2c[pl.ds(base, _B), :]
        ac = (jnp.maximum(cx2 - cx1, 0.0) * jnp.maximum(cy2 - cy1, 0.0))

        rx1 = x1r[pl.ds(b, 1), :]
        ry1 = y1r[pl.ds(b, 1), :]
        rx2 = x2r[pl.ds(b, 1), :]
        ry2 = y2r[pl.ds(b, 1), :]
        ar = (jnp.maximum(rx2 - rx1, 0.0) * jnp.maximum(ry2 - ry1, 0.0))

        w = jnp.maximum(jnp.minimum(cx2, rx2) - jnp.maximum(cx1, rx1), 0.0)
        h = jnp.maximum(jnp.minimum(cy2, ry2) - jnp.maximum(cy1, ry1), 0.0)
        inter = w * h
        union = ac + ar - inter
        iou = inter / jnp.maximum(union, 1e-8)
        m_scr[:, :] = jnp.where((iou > _TH) & jlt, 1.0, 0.0)

        rem0 = pltpu.roll(cross_ref[:, :], (_NP - base) % _NP, 1)[:, 0:_B]

        def group(g, rem_rot):
            mg = pltpu.roll(m_scr[pl.ds(g * 8, 8), :], (_B - g * 8) % _B, 1)
            for k in range(8):
                row = mg[k:k + 1, :]
                flag = rem_rot[:, k:k + 1]
                rem_rot = jnp.maximum(rem_rot, row * (1.0 - flag))
            return pltpu.roll(rem_rot, _B - 8, 1)

        rem = jax.lax.fori_loop(0, _B // 8, group, rem0)
        out_ref[pl.ds(b, 1), :] = rem
        keep = 1.0 - rem

        W = jnp.maximum(jnp.minimum(cx2, VX2) - jnp.maximum(cx1, VX1), 0.0)
        H = jnp.maximum(jnp.minimum(cy2, VY2) - jnp.maximum(cy1, VY1), 0.0)
        I = W * H
        U = ac + AV - I
        IO = I / jnp.maximum(U, 1e-8)
        MK = jnp.where(IO > _TH, 1.0, 0.0)
        counts = jax.lax.dot_general(
            keep, MK, (((1,), (0,)), ((), ())),
            preferred_element_type=jnp.float32)
        sup = jnp.where((counts >= 0.5) & (gl >= base + _B), 1.0, 0.0)
        cross_ref[:, :] = jnp.maximum(cross_ref[:, :], sup)
        return carry

    jax.lax.fori_loop(0, _NB, blk, 0)


def kernel(boxes, scores):
    order = jnp.argsort(-scores)
    b = jnp.take(boxes, order, axis=0)
    s = jnp.take(scores, order)
    bp = jnp.pad(b, ((0, _NP - _N), (0, 0)))
    x1c = bp[:, 0:1]
    y1c = bp[:, 1:2]
    x2c = bp[:, 2:3]
    y2c = bp[:, 3:4]
    x1r = bp[:, 0].reshape(_NB, _B)
    y1r = bp[:, 1].reshape(_NB, _B)
    x2r = bp[:, 2].reshape(_NB, _B)
    y2r = bp[:, 3].reshape(_NB, _B)
    X1 = bp[:, 0].reshape(1, _NP)
    Y1 = bp[:, 1].reshape(1, _NP)
    X2 = bp[:, 2].reshape(1, _NP)
    Y2 = bp[:, 3].reshape(1, _NP)
    removed = pl.pallas_call(
        _nms_body,
        out_shape=jax.ShapeDtypeStruct((_NB, _B), jnp.float32),
        scratch_shapes=[pltpu.VMEM((_B, _B), jnp.float32),
                        pltpu.VMEM((1, _NP), jnp.float32)],
    )(x1c, y1c, x2c, y2c, x1r, y1r, x2r, y2r, X1, Y1, X2, Y2)
    keep = 1.0 - removed.reshape(_NP)[:_N]
    proposals = jnp.concatenate([b, s[:, None]], axis=1) * keep[:, None]
    return proposals


# speedup vs baseline: 2.0006x; 1.1958x over previous
import jax
import jax.numpy as jnp
from jax.experimental import pallas as pl
from jax.experimental.pallas import tpu as pltpu

_N = 5000
_B = 128
_NB = 40
_NP = _NB * _B
_TH = 0.7


def _nms_body(x1c, y1c, x2c, y2c, x1r, y1r, x2r, y2r,
              X1, Y1, X2, Y2, out_ref, cross_ref):
    out_ref[:, :] = jnp.zeros((_NB, _B), jnp.float32)
    cross_ref[:, :] = jnp.zeros((1, _NP), jnp.float32)
    gl = jax.lax.broadcasted_iota(jnp.int32, (1, _NP), 1)
    jlt = (jax.lax.broadcasted_iota(jnp.int32, (_B, _B), 0)
           < jax.lax.broadcasted_iota(jnp.int32, (_B, _B), 1))
    VX1 = X1[:, :]
    VY1 = Y1[:, :]
    VX2 = X2[:, :]
    VY2 = Y2[:, :]
    AV = (jnp.maximum(VX2 - VX1, 0.0) * jnp.maximum(VY2 - VY1, 0.0))

    def blk(b, carry):
        base = b * _B
        cx1 = x1c[pl.ds(base, _B), :]
        cy1 = y1c[pl.ds(base, _B), :]
        cx2 = x2c[pl.ds(base, _B), :]
        cy2 = y2c[pl.ds(base, _B), :]
        ac = (jnp.maximum(cx2 - cx1, 0.0) * jnp.maximum(cy2 - cy1, 0.0))

        rx1 = x1r[pl.ds(b, 1), :]
        ry1 = y1r[pl.ds(b, 1), :]
        rx2 = x2r[pl.ds(b, 1), :]
        ry2 = y2r[pl.ds(b, 1), :]
        ar = (jnp.maximum(rx2 - rx1, 0.0) * jnp.maximum(ry2 - ry1, 0.0))

        w = jnp.maximum(jnp.minimum(cx2, rx2) - jnp.maximum(cx1, rx1), 0.0)
        h = jnp.maximum(jnp.minimum(cy2, ry2) - jnp.maximum(cy1, ry1), 0.0)
        inter = w * h
        union = ac + ar - inter
        iou = inter / jnp.maximum(union, 1e-8)
        m = jnp.where((iou > _TH) & jlt, 1.0, 0.0)

        rem0 = pltpu.roll(cross_ref[:, :], (_NP - base) % _NP, 1)[:, 0:_B]

        rem = rem0
        for k in range(_B):
            flag = rem[:, k:k + 1]
            rem = jnp.maximum(rem, m[k:k + 1, :] * (1.0 - flag))
        out_ref[pl.ds(b, 1), :] = rem
        keep = 1.0 - rem

        W = jnp.maximum(jnp.minimum(cx2, VX2) - jnp.maximum(cx1, VX1), 0.0)
        H = jnp.maximum(jnp.minimum(cy2, VY2) - jnp.maximum(cy1, VY1), 0.0)
        I = W * H
        U = ac + AV - I
        IO = I / jnp.maximum(U, 1e-8)
        MK = jnp.where(IO > _TH, 1.0, 0.0)
        counts = jax.lax.dot_general(
            keep, MK, (((1,), (0,)), ((), ())),
            preferred_element_type=jnp.float32)
        sup = jnp.where((counts >= 0.5) & (gl >= base + _B), 1.0, 0.0)
        cross_ref[:, :] = jnp.maximum(cross_ref[:, :], sup)
        return carry

    jax.lax.fori_loop(0, _NB, blk, 0)


def kernel(boxes, scores):
    order = jnp.argsort(-scores)
    b = jnp.take(boxes, order, axis=0)
    s = jnp.take(scores, order)
    bp = jnp.pad(b, ((0, _NP - _N), (0, 0)))
    x1c = bp[:, 0:1]
    y1c = bp[:, 1:2]
    x2c = bp[:, 2:3]
    y2c = bp[:, 3:4]
    x1r = bp[:, 0].reshape(_NB, _B)
    y1r = bp[:, 1].reshape(_NB, _B)
    x2r = bp[:, 2].reshape(_NB, _B)
    y2r = bp[:, 3].reshape(_NB, _B)
    X1 = bp[:, 0].reshape(1, _NP)
    Y1 = bp[:, 1].reshape(1, _NP)
    X2 = bp[:, 2].reshape(1, _NP)
    Y2 = bp[:, 3].reshape(1, _NP)
    removed = pl.pallas_call(
        _nms_body,
        out_shape=jax.ShapeDtypeStruct((_NB, _B), jnp.float32),
        scratch_shapes=[pltpu.VMEM((1, _NP), jnp.float32)],
    )(x1c, y1c, x2c, y2c, x1r, y1r, x2r, y2r, X1, Y1, X2, Y2)
    keep = 1.0 - removed.reshape(_NP)[:_N]
    proposals = jnp.concatenate([b, s[:, None]], axis=1) * keep[:, None]
    return proposals


# speedup vs baseline: 6.4190x; 3.2086x over previous
import jax
import jax.numpy as jnp
from jax.experimental import pallas as pl
from jax.experimental.pallas import tpu as pltpu

_N = 5000
_B = 128
_NB = 40
_NP = _NB * _B
_TH = 0.7


def _nms_body(x1c, y1c, x2c, y2c, x1r, y1r, x2r, y2r,
              X1, Y1, X2, Y2, out_ref, cross_ref):
    out_ref[:, :] = jnp.zeros((_NB, _B), jnp.float32)
    cross_ref[:, :] = jnp.zeros((1, _NP), jnp.float32)
    gl = jax.lax.broadcasted_iota(jnp.int32, (1, _NP), 1)
    jlt = (jax.lax.broadcasted_iota(jnp.int32, (_B, _B), 0)
           < jax.lax.broadcasted_iota(jnp.int32, (_B, _B), 1))
    VX1 = X1[:, :]
    VY1 = Y1[:, :]
    VX2 = X2[:, :]
    VY2 = Y2[:, :]
    AV = (jnp.maximum(VX2 - VX1, 0.0) * jnp.maximum(VY2 - VY1, 0.0))

    def blk(b, carry):
        base = b * _B
        cx1 = x1c[pl.ds(base, _B), :]
        cy1 = y1c[pl.ds(base, _B), :]
        cx2 = x2c[pl.ds(base, _B), :]
        cy2 = y2c[pl.ds(base, _B), :]
        ac = (jnp.maximum(cx2 - cx1, 0.0) * jnp.maximum(cy2 - cy1, 0.0))

        rx1 = x1r[pl.ds(b, 1), :]
        ry1 = y1r[pl.ds(b, 1), :]
        rx2 = x2r[pl.ds(b, 1), :]
        ry2 = y2r[pl.ds(b, 1), :]
        ar = (jnp.maximum(rx2 - rx1, 0.0) * jnp.maximum(ry2 - ry1, 0.0))

        w = jnp.maximum(jnp.minimum(cx2, rx2) - jnp.maximum(cx1, rx1), 0.0)
        h = jnp.maximum(jnp.minimum(cy2, ry2) - jnp.maximum(cy1, ry1), 0.0)
        inter = w * h
        union = ac + ar - inter
        iou = inter / jnp.maximum(union, 1e-8)
        m = jnp.where((iou > _TH) & jlt, 1.0, 0.0)

        rem0 = pltpu.roll(cross_ref[:, :], (_NP - base) % _NP, 1)[:, 0:_B]

        def fp_cond(carry):
            return carry[1]

        def fp_body(carry):
            rem, _ = carry
            counts = jax.lax.dot_general(
                1.0 - rem, m, (((1,), (0,)), ((), ())),
                preferred_element_type=jnp.float32)
            rem_new = jnp.maximum(rem0, jnp.where(counts >= 0.5, 1.0, 0.0))
            changed = jnp.sum(jnp.abs(rem_new - rem)) > 0.0
            return (rem_new, changed)

        rem, _ = jax.lax.while_loop(fp_cond, fp_body, (rem0, True))
        out_ref[pl.ds(b, 1), :] = rem
        keep = 1.0 - rem

        W = jnp.maximum(jnp.minimum(cx2, VX2) - jnp.maximum(cx1, VX1), 0.0)
        H = jnp.maximum(jnp.minimum(cy2, VY2) - jnp.maximum(cy1, VY1), 0.0)
        I = W * H
        U = ac + AV - I
        IO = I / jnp.maximum(U, 1e-8)
        MK = jnp.where(IO > _TH, 1.0, 0.0)
        counts = jax.lax.dot_general(
            keep, MK, (((1,), (0,)), ((), ())),
            preferred_element_type=jnp.float32)
        sup = jnp.where((counts >= 0.5) & (gl >= base + _B), 1.0, 0.0)
        cross_ref[:, :] = jnp.maximum(cross_ref[:, :], sup)
        return carry

    jax.lax.fori_loop(0, _NB, blk, 0)


def kernel(boxes, scores):
    order = jnp.argsort(-scores)
    b = jnp.take(boxes, order, axis=0)
    s = jnp.take(scores, order)
    bp = jnp.pad(b, ((0, _NP - _N), (0, 0)))
    x1c = bp[:, 0:1]
    y1c = bp[:, 1:2]
    x2c = bp[:, 2:3]
    y2c = bp[:, 3:4]
    x1r = bp[:, 0].reshape(_NB, _B)
    y1r = bp[:, 1].reshape(_NB, _B)
    x2r = bp[:, 2].reshape(_NB, _B)
    y2r = bp[:, 3].reshape(_NB, _B)
    X1 = bp[:, 0].reshape(1, _NP)
    Y1 = bp[:, 1].reshape(1, _NP)
    X2 = bp[:, 2].reshape(1, _NP)
    Y2 = bp[:, 3].reshape(1, _NP)
    removed = pl.pallas_call(
        _nms_body,
        out_shape=jax.ShapeDtypeStruct((_NB, _B), jnp.float32),
        scratch_shapes=[pltpu.VMEM((1, _NP), jnp.float32)],
    )(x1c, y1c, x2c, y2c, x1r, y1r, x2r, y2r, X1, Y1, X2, Y2)
    keep = 1.0 - removed.reshape(_NP)[:_N]
    proposals = jnp.concatenate([b, s[:, None]], axis=1) * keep[:, None]
    return proposals


# speedup vs baseline: 7.2077x; 1.1229x over previous
import jax
import jax.numpy as jnp
from jax.experimental import pallas as pl
from jax.experimental.pallas import tpu as pltpu

_N = 5000
_B = 128
_NB = 40
_NP = _NB * _B
_NQ = _NP // 4
_TH = 0.7


def _nms_body(x1c, y1c, x2c, y2c, x1r, y1r, x2r, y2r,
              X1, Y1, X2, Y2, out_ref, cross_ref):
    out_ref[:, :] = jnp.zeros((_NB, _B), jnp.float32)
    cross_ref[:, :] = jnp.zeros((1, _NP), jnp.float32)
    gl = jax.lax.broadcasted_iota(jnp.int32, (1, _NP), 1)
    jlt = (jax.lax.broadcasted_iota(jnp.int32, (_B, _B), 0)
           < jax.lax.broadcasted_iota(jnp.int32, (_B, _B), 1))
    VX1 = X1[:, :]
    VY1 = Y1[:, :]
    VX2 = X2[:, :]
    VY2 = Y2[:, :]
    AV = (jnp.maximum(VX2 - VX1, 0.0) * jnp.maximum(VY2 - VY1, 0.0))

    def blk(b, carry):
        base = b * _B
        cx1 = x1c[pl.ds(base, _B), :]
        cy1 = y1c[pl.ds(base, _B), :]
        cx2 = x2c[pl.ds(base, _B), :]
        cy2 = y2c[pl.ds(base, _B), :]
        ac = (jnp.maximum(cx2 - cx1, 0.0) * jnp.maximum(cy2 - cy1, 0.0))

        rx1 = x1r[pl.ds(b, 1), :]
        ry1 = y1r[pl.ds(b, 1), :]
        rx2 = x2r[pl.ds(b, 1), :]
        ry2 = y2r[pl.ds(b, 1), :]
        ar = (jnp.maximum(rx2 - rx1, 0.0) * jnp.maximum(ry2 - ry1, 0.0))

        w = jnp.maximum(jnp.minimum(cx2, rx2) - jnp.maximum(cx1, rx1), 0.0)
        h = jnp.maximum(jnp.minimum(cy2, ry2) - jnp.maximum(cy1, ry1), 0.0)
        inter = w * h
        union = ac + ar - inter
        iou = inter / jnp.maximum(union, 1e-8)
        m = jnp.where((iou > _TH) & jlt, 1.0, 0.0)

        rem0 = pltpu.roll(cross_ref[:, :], (_NP - base) % _NP, 1)[:, 0:_B]

        def fp_cond(carry):
            return carry[1]

        def fp_step(rem):
            counts = jax.lax.dot_general(
                1.0 - rem, m, (((1,), (0,)), ((), ())),
                preferred_element_type=jnp.float32)
            return jnp.maximum(rem0, jnp.where(counts >= 0.5, 1.0, 0.0))

        def fp_body(carry):
            rem, _ = carry
            rem_new = fp_step(fp_step(rem))
            changed = jnp.sum(jnp.abs(rem_new - rem)) > 0.0
            return (rem_new, changed)

        rem, _ = jax.lax.while_loop(fp_cond, fp_body, (rem0, True))
        out_ref[pl.ds(b, 1), :] = rem
        keep = 1.0 - rem

        for q in range(4):
            lo = q * _NQ
            @pl.when((q + 1) * _NQ > base + _B)
            def _():
                vx1 = VX1[:, lo:lo + _NQ]
                vy1 = VY1[:, lo:lo + _NQ]
                vx2 = VX2[:, lo:lo + _NQ]
                vy2 = VY2[:, lo:lo + _NQ]
                W = jnp.maximum(jnp.minimum(cx2, vx2) - jnp.maximum(cx1, vx1), 0.0)
                H = jnp.maximum(jnp.minimum(cy2, vy2) - jnp.maximum(cy1, vy1), 0.0)
                I = W * H
                U = ac + AV[:, lo:lo + _NQ] - I
                IO = I / jnp.maximum(U, 1e-8)
                MK = jnp.where(IO > _TH, 1.0, 0.0)
                counts = jax.lax.dot_general(
                    keep, MK, (((1,), (0,)), ((), ())),
                    preferred_element_type=jnp.float32)
                sup = jnp.where(
                    (counts >= 0.5) & (gl[:, lo:lo + _NQ] >= base + _B),
                    1.0, 0.0)
                cross_ref[:, lo:lo + _NQ] = jnp.maximum(
                    cross_ref[:, lo:lo + _NQ], sup)
        return carry

    jax.lax.fori_loop(0, _NB, blk, 0)


def kernel(boxes, scores):
    order = jnp.argsort(-scores)
    sa = jnp.take(jnp.concatenate([boxes, scores[:, None]], axis=1),
                  order, axis=0)
    b = sa[:, :4]
    bp = jnp.pad(b, ((0, _NP - _N), (0, 0)))
    x1c = bp[:, 0:1]
    y1c = bp[:, 1:2]
    x2c = bp[:, 2:3]
    y2c = bp[:, 3:4]
    x1r = bp[:, 0].reshape(_NB, _B)
    y1r = bp[:, 1].reshape(_NB, _B)
    x2r = bp[:, 2].reshape(_NB, _B)
    y2r = bp[:, 3].reshape(_NB, _B)
    X1 = bp[:, 0].reshape(1, _NP)
    Y1 = bp[:, 1].reshape(1, _NP)
    X2 = bp[:, 2].reshape(1, _NP)
    Y2 = bp[:, 3].reshape(1, _NP)
    removed = pl.pallas_call(
        _nms_body,
        out_shape=jax.ShapeDtypeStruct((_NB, _B), jnp.float32),
        scratch_shapes=[pltpu.VMEM((1, _NP), jnp.float32)],
    )(x1c, y1c, x2c, y2c, x1r, y1r, x2r, y2r, X1, Y1, X2, Y2)
    keep = 1.0 - removed.reshape(_NP)[:_N]
    return sa * keep[:, None]


# speedup vs baseline: 8.2864x; 1.1497x over previous
import jax
import jax.numpy as jnp
from jax.experimental import pallas as pl
from jax.experimental.pallas import tpu as pltpu

_N = 5000
_B = 256
_NB = 20
_NP = _NB * _B
_NQ = _NP // 4
_TH = 0.7


def _nms_body(x1c, y1c, x2c, y2c, x1r, y1r, x2r, y2r,
              X1, Y1, X2, Y2, out_ref, cross_ref):
    out_ref[:, :] = jnp.zeros((_NB, _B), jnp.float32)
    cross_ref[:, :] = jnp.zeros((1, _NP), jnp.float32)
    gl = jax.lax.broadcasted_iota(jnp.int32, (1, _NP), 1)
    jlt = (jax.lax.broadcasted_iota(jnp.int32, (_B, _B), 0)
           < jax.lax.broadcasted_iota(jnp.int32, (_B, _B), 1))
    VX1 = X1[:, :]
    VY1 = Y1[:, :]
    VX2 = X2[:, :]
    VY2 = Y2[:, :]
    AV = (jnp.maximum(VX2 - VX1, 0.0) * jnp.maximum(VY2 - VY1, 0.0))

    def blk(b, carry):
        base = b * _B
        cx1 = x1c[pl.ds(base, _B), :]
        cy1 = y1c[pl.ds(base, _B), :]
        cx2 = x2c[pl.ds(base, _B), :]
        cy2 = y2c[pl.ds(base, _B), :]
        ac = (jnp.maximum(cx2 - cx1, 0.0) * jnp.maximum(cy2 - cy1, 0.0))

        rx1 = x1r[pl.ds(b, 1), :]
        ry1 = y1r[pl.ds(b, 1), :]
        rx2 = x2r[pl.ds(b, 1), :]
        ry2 = y2r[pl.ds(b, 1), :]
        ar = (jnp.maximum(rx2 - rx1, 0.0) * jnp.maximum(ry2 - ry1, 0.0))

        w = jnp.maximum(jnp.minimum(cx2, rx2) - jnp.maximum(cx1, rx1), 0.0)
        h = jnp.maximum(jnp.minimum(cy2, ry2) - jnp.maximum(cy1, ry1), 0.0)
        inter = w * h
        union = ac + ar - inter
        iou = inter / jnp.maximum(union, 1e-8)
        m = jnp.where((iou > _TH) & jlt, 1.0, 0.0)

        rem0 = pltpu.roll(cross_ref[:, :], (_NP - base) % _NP, 1)[:, 0:_B]

        def fp_cond(carry):
            return carry[1]

        def fp_step(rem):
            counts = jax.lax.dot_general(
                1.0 - rem, m, (((1,), (0,)), ((), ())),
                preferred_element_type=jnp.float32)
            return jnp.maximum(rem0, jnp.where(counts >= 0.5, 1.0, 0.0))

        def fp_body(carry):
            rem, _ = carry
            rem_new = fp_step(fp_step(rem))
            changed = jnp.sum(jnp.abs(rem_new - rem)) > 0.0
            return (rem_new, changed)

        rem, _ = jax.lax.while_loop(fp_cond, fp_body, (rem0, True))
        out_ref[pl.ds(b, 1), :] = rem
        keep = 1.0 - rem

        for q in range(4):
            lo = q * _NQ
            @pl.when((q + 1) * _NQ > base + _B)
            def _():
                vx1 = VX1[:, lo:lo + _NQ]
                vy1 = VY1[:, lo:lo + _NQ]
                vx2 = VX2[:, lo:lo + _NQ]
                vy2 = VY2[:, lo:lo + _NQ]
                W = jnp.maximum(jnp.minimum(cx2, vx2) - jnp.maximum(cx1, vx1), 0.0)
                H = jnp.maximum(jnp.minimum(cy2, vy2) - jnp.maximum(cy1, vy1), 0.0)
                I = W * H
                U = ac + AV[:, lo:lo + _NQ] - I
                IO = I / jnp.maximum(U, 1e-8)
                MK = jnp.where(IO > _TH, 1.0, 0.0)
                counts = jax.lax.dot_general(
                    keep, MK, (((1,), (0,)), ((), ())),
                    preferred_element_type=jnp.float32)
                sup = jnp.where(
                    (counts >= 0.5) & (gl[:, lo:lo + _NQ] >= base + _B),
                    1.0, 0.0)
                cross_ref[:, lo:lo + _NQ] = jnp.maximum(
                    cross_ref[:, lo:lo + _NQ], sup)
        return carry

    jax.lax.fori_loop(0, _NB, blk, 0)


def kernel(boxes, scores):
    order = jnp.argsort(-scores)
    sa = jnp.take(jnp.concatenate([boxes, scores[:, None]], axis=1),
                  order, axis=0)
    b = sa[:, :4]
    bp = jnp.pad(b, ((0, _NP - _N), (0, 0)))
    x1c = bp[:, 0:1]
    y1c = bp[:, 1:2]
    x2c = bp[:, 2:3]
    y2c = bp[:, 3:4]
    x1r = bp[:, 0].reshape(_NB, _B)
    y1r = bp[:, 1].reshape(_NB, _B)
    x2r = bp[:, 2].reshape(_NB, _B)
    y2r = bp[:, 3].reshape(_NB, _B)
    X1 = bp[:, 0].reshape(1, _NP)
    Y1 = bp[:, 1].reshape(1, _NP)
    X2 = bp[:, 2].reshape(1, _NP)
    Y2 = bp[:, 3].reshape(1, _NP)
    removed = pl.pallas_call(
        _nms_body,
        out_shape=jax.ShapeDtypeStruct((_NB, _B), jnp.float32),
        scratch_shapes=[pltpu.VMEM((1, _NP), jnp.float32)],
    )(x1c, y1c, x2c, y2c, x1r, y1r, x2r, y2r, X1, Y1, X2, Y2)
    keep = 1.0 - removed.reshape(_NP)[:_N]
    return sa * keep[:, None]


# speedup vs baseline: 8.8742x; 1.0709x over previous
import jax
import jax.numpy as jnp
from jax.experimental import pallas as pl
from jax.experimental.pallas import tpu as pltpu

_N = 5000
_B = 512
_NB = 10
_NP = _NB * _B
_NQ = _NP // 4
_TH = 0.7


def _nms_body(x1c, y1c, x2c, y2c, x1r, y1r, x2r, y2r,
              X1, Y1, X2, Y2, out_ref, cross_ref):
    out_ref[:, :] = jnp.zeros((_NB, _B), jnp.float32)
    cross_ref[:, :] = jnp.zeros((1, _NP), jnp.float32)
    gl = jax.lax.broadcasted_iota(jnp.int32, (1, _NP), 1)
    jlt = (jax.lax.broadcasted_iota(jnp.int32, (_B, _B), 0)
           < jax.lax.broadcasted_iota(jnp.int32, (_B, _B), 1))
    VX1 = X1[:, :]
    VY1 = Y1[:, :]
    VX2 = X2[:, :]
    VY2 = Y2[:, :]
    AV = (jnp.maximum(VX2 - VX1, 0.0) * jnp.maximum(VY2 - VY1, 0.0))

    def blk(b, carry):
        base = b * _B
        cx1 = x1c[pl.ds(base, _B), :]
        cy1 = y1c[pl.ds(base, _B), :]
        cx2 = x2c[pl.ds(base, _B), :]
        cy2 = y2c[pl.ds(base, _B), :]
        ac = (jnp.maximum(cx2 - cx1, 0.0) * jnp.maximum(cy2 - cy1, 0.0))

        rx1 = x1r[pl.ds(b, 1), :]
        ry1 = y1r[pl.ds(b, 1), :]
        rx2 = x2r[pl.ds(b, 1), :]
        ry2 = y2r[pl.ds(b, 1), :]
        ar = (jnp.maximum(rx2 - rx1, 0.0) * jnp.maximum(ry2 - ry1, 0.0))

        w = jnp.maximum(jnp.minimum(cx2, rx2) - jnp.maximum(cx1, rx1), 0.0)
        h = jnp.maximum(jnp.minimum(cy2, ry2) - jnp.maximum(cy1, ry1), 0.0)
        inter = w * h
        union = ac + ar - inter
        iou = inter / jnp.maximum(union, 1e-8)
        m = jnp.where((iou > _TH) & jlt, 1.0, 0.0)

        rem0 = pltpu.roll(cross_ref[:, :], (_NP - base) % _NP, 1)[:, 0:_B]

        def fp_cond(carry):
            return carry[1]

        def fp_step(rem):
            counts = jax.lax.dot_general(
                1.0 - rem, m, (((1,), (0,)), ((), ())),
                preferred_element_type=jnp.float32)
            return jnp.maximum(rem0, jnp.where(counts >= 0.5, 1.0, 0.0))

        def fp_body(carry):
            rem, _ = carry
            rem_new = fp_step(fp_step(rem))
            changed = jnp.sum(jnp.abs(rem_new - rem)) > 0.0
            return (rem_new, changed)

        rem, _ = jax.lax.while_loop(fp_cond, fp_body, (rem0, True))
        out_ref[pl.ds(b, 1), :] = rem
        keep = 1.0 - rem

        for q in range(4):
            lo = q * _NQ
            @pl.when((q + 1) * _NQ > base + _B)
            def _():
                vx1 = VX1[:, lo:lo + _NQ]
                vy1 = VY1[:, lo:lo + _NQ]
                vx2 = VX2[:, lo:lo + _NQ]
                vy2 = VY2[:, lo:lo + _NQ]
                W = jnp.maximum(jnp.minimum(cx2, vx2) - jnp.maximum(cx1, vx1), 0.0)
                H = jnp.maximum(jnp.minimum(cy2, vy2) - jnp.maximum(cy1, vy1), 0.0)
                I = W * H
                U = ac + AV[:, lo:lo + _NQ] - I
                IO = I / jnp.maximum(U, 1e-8)
                MK = jnp.where(IO > _TH, 1.0, 0.0)
                counts = jax.lax.dot_general(
                    keep, MK, (((1,), (0,)), ((), ())),
                    preferred_element_type=jnp.float32)
                sup = jnp.where(
                    (counts >= 0.5) & (gl[:, lo:lo + _NQ] >= base + _B),
                    1.0, 0.0)
                cross_ref[:, lo:lo + _NQ] = jnp.maximum(
                    cross_ref[:, lo:lo + _NQ], sup)
        return carry

    jax.lax.fori_loop(0, _NB, blk, 0)


def kernel(boxes, scores):
    order = jnp.argsort(-scores)
    sa = jnp.take(jnp.concatenate([boxes, scores[:, None]], axis=1),
                  order, axis=0)
    b = sa[:, :4]
    bp = jnp.pad(b, ((0, _NP - _N), (0, 0)))
    x1c = bp[:, 0:1]
    y1c = bp[:, 1:2]
    x2c = bp[:, 2:3]
    y2c = bp[:, 3:4]
    x1r = bp[:, 0].reshape(_NB, _B)
    y1r = bp[:, 1].reshape(_NB, _B)
    x2r = bp[:, 2].reshape(_NB, _B)
    y2r = bp[:, 3].reshape(_NB, _B)
    X1 = bp[:, 0].reshape(1, _NP)
    Y1 = bp[:, 1].reshape(1, _NP)
    X2 = bp[:, 2].reshape(1, _NP)
    Y2 = bp[:, 3].reshape(1, _NP)
    removed = pl.pallas_call(
        _nms_body,
        out_shape=jax.ShapeDtypeStruct((_NB, _B), jnp.float32),
        scratch_shapes=[pltpu.VMEM((1, _NP), jnp.float32)],
    )(x1c, y1c, x2c, y2c, x1r, y1r, x2r, y2r, X1, Y1, X2, Y2)
    keep = 1.0 - removed.reshape(_NP)[:_N]
    return sa * keep[:, None]


# speedup vs baseline: 12.4609x; 1.4042x over previous
import jax
import jax.numpy as jnp
from jax.experimental import pallas as pl
from jax.experimental.pallas import tpu as pltpu

_N = 5000
_B = 512
_NB = 10
_NP = _NB * _B
_NQ = _NP // 4
_TH = 0.7


def _nms_body(x1c, y1c, x2c, y2c, x1r, y1r, x2r, y2r,
              X1, Y1, X2, Y2, out_ref, cross_ref):
    out_ref[:, :] = jnp.zeros((_NB, _B), jnp.float32)
    cross_ref[:, :] = jnp.zeros((1, _NP), jnp.float32)
    gl = jax.lax.broadcasted_iota(jnp.int32, (1, _NP), 1)
    jlt = (jax.lax.broadcasted_iota(jnp.int32, (_B, _B), 0)
           < jax.lax.broadcasted_iota(jnp.int32, (_B, _B), 1))
    VX1 = X1[:, :]
    VY1 = Y1[:, :]
    VX2 = X2[:, :]
    VY2 = Y2[:, :]
    AV = (jnp.maximum(VX2 - VX1, 0.0) * jnp.maximum(VY2 - VY1, 0.0))

    def blk(b, carry):
        base = b * _B
        cx1 = x1c[pl.ds(base, _B), :]
        cy1 = y1c[pl.ds(base, _B), :]
        cx2 = x2c[pl.ds(base, _B), :]
        cy2 = y2c[pl.ds(base, _B), :]
        ac = (jnp.maximum(cx2 - cx1, 0.0) * jnp.maximum(cy2 - cy1, 0.0))

        rx1 = x1r[pl.ds(b, 1), :]
        ry1 = y1r[pl.ds(b, 1), :]
        rx2 = x2r[pl.ds(b, 1), :]
        ry2 = y2r[pl.ds(b, 1), :]
        ar = (jnp.maximum(rx2 - rx1, 0.0) * jnp.maximum(ry2 - ry1, 0.0))

        w = jnp.maximum(jnp.minimum(cx2, rx2) - jnp.maximum(cx1, rx1), 0.0)
        h = jnp.maximum(jnp.minimum(cy2, ry2) - jnp.maximum(cy1, ry1), 0.0)
        inter = w * h
        union = ac + ar - inter
        iou = inter / jnp.maximum(union, 1e-8)
        m = jnp.where((iou > _TH) & jlt, 1.0, 0.0)

        rem0 = pltpu.roll(cross_ref[:, :], (_NP - base) % _NP, 1)[:, 0:_B]

        def fp_cond(carry):
            return carry[1]

        def fp_step(rem):
            counts = jax.lax.dot_general(
                1.0 - rem, m, (((1,), (0,)), ((), ())),
                preferred_element_type=jnp.float32)
            return jnp.maximum(rem0, jnp.where(counts >= 0.5, 1.0, 0.0))

        def fp_body(carry):
            rem, _ = carry
            rem_new = fp_step(fp_step(rem))
            changed = jnp.sum(jnp.abs(rem_new - rem)) > 0.0
            return (rem_new, changed)

        rem, _ = jax.lax.while_loop(fp_cond, fp_body, (rem0, True))
        out_ref[pl.ds(b, 1), :] = rem
        keep = 1.0 - rem

        for q in range(4):
            lo = q * _NQ
            @pl.when((q + 1) * _NQ > base + _B)
            def _():
                vx1 = VX1[:, lo:lo + _NQ]
                vy1 = VY1[:, lo:lo + _NQ]
                vx2 = VX2[:, lo:lo + _NQ]
                vy2 = VY2[:, lo:lo + _NQ]
                W = jnp.maximum(jnp.minimum(cx2, vx2) - jnp.maximum(cx1, vx1), 0.0)
                H = jnp.maximum(jnp.minimum(cy2, vy2) - jnp.maximum(cy1, vy1), 0.0)
                I = W * H
                U = ac + AV[:, lo:lo + _NQ] - I
                IO = I / jnp.maximum(U, 1e-8)
                MK = jnp.where(IO > _TH, 1.0, 0.0)
                counts = jax.lax.dot_general(
                    keep, MK, (((1,), (0,)), ((), ())),
                    preferred_element_type=jnp.float32)
                sup = jnp.where(
                    (counts >= 0.5) & (gl[:, lo:lo + _NQ] >= base + _B),
                    1.0, 0.0)
                cross_ref[:, lo:lo + _NQ] = jnp.maximum(
                    cross_ref[:, lo:lo + _NQ], sup)
        return carry

    jax.lax.fori_loop(0, _NB, blk, 0)


def kernel(boxes, scores):
    _, sx1, sy1, sx2, sy2, ss = jax.lax.sort(
        (-scores, boxes[:, 0], boxes[:, 1], boxes[:, 2], boxes[:, 3], scores),
        num_keys=1, is_stable=True)
    sa = jnp.stack([sx1, sy1, sx2, sy2, ss], axis=1)
    pad = (0, _NP - _N)
    px1 = jnp.pad(sx1, pad)
    py1 = jnp.pad(sy1, pad)
    px2 = jnp.pad(sx2, pad)
    py2 = jnp.pad(sy2, pad)
    x1c = px1[:, None]
    y1c = py1[:, None]
    x2c = px2[:, None]
    y2c = py2[:, None]
    x1r = px1.reshape(_NB, _B)
    y1r = py1.reshape(_NB, _B)
    x2r = px2.reshape(_NB, _B)
    y2r = py2.reshape(_NB, _B)
    X1 = px1.reshape(1, _NP)
    Y1 = py1.reshape(1, _NP)
    X2 = px2.reshape(1, _NP)
    Y2 = py2.reshape(1, _NP)
    removed = pl.pallas_call(
        _nms_body,
        out_shape=jax.ShapeDtypeStruct((_NB, _B), jnp.float32),
        scratch_shapes=[pltpu.VMEM((1, _NP), jnp.float32)],
    )(x1c, y1c, x2c, y2c, x1r, y1r, x2r, y2r, X1, Y1, X2, Y2)
    keep = 1.0 - removed.reshape(_NP)[:_N]
    return sa * keep[:, None]
